# baseline (speedup 1.0000x reference)
import functools

import jax
import jax.numpy as jnp
from jax import lax
from jax.experimental import pallas as pl
from jax.experimental.pallas import tpu as pltpu
from jax.experimental.pallas import tpu_sc as plsc

B = 8
TV, DV = 64, 512
TP, DP = 256, 64
TI, DI = 512, 32
TT = 128
NPTS = 262144
GRID = 64
INV_VOX = 16.0
GMIN = -2.0

NW = 32
B_PER_SC = 4
HIST_N = B_PER_SC * GRID * GRID * GRID
DUMP = HIST_N
PTS_PER_TILE = (B_PER_SC * NPTS) // 16
CHUNK_PTS = 4096
N_CHUNKS = PTS_PER_TILE // CHUNK_PTS
RED_WORDS = HIST_N // 16
RED_CHUNK = 16384

G_CHUNK = 32768
N_G = NPTS // G_CHUNK


def _idx_body(x_ref, y_ref, z_ref, out_ref):
    def coords(ref):
        cf = (ref[0] - GMIN) * INV_VOX
        ci = jnp.minimum(jnp.maximum(cf, 0.0), 63.0).astype(jnp.int32)
        ok = (cf >= 0.0) & (cf < float(GRID))
        return ci, ok

    cx, okx = coords(x_ref)
    cy, oky = coords(y_ref)
    cz, okz = coords(z_ref)
    valid = okx & oky & okz
    bloc = lax.broadcasted_iota(jnp.int32, (B, G_CHUNK), 0) % B_PER_SC
    spread = lax.broadcasted_iota(jnp.int32, (B, G_CHUNK), 1) % 128
    flat = bloc * (GRID * GRID * GRID) + cx * (GRID * GRID) + cy * GRID + cz
    idx = jnp.where(valid, flat, DUMP + spread)
    out_ref[...] = idx.reshape(B * G_CHUNK)


def _point_bin_indices(points):
    pts3 = jnp.transpose(points, (2, 0, 1))
    idx = pl.pallas_call(
        _idx_body,
        grid=(N_G,),
        in_specs=[
            pl.BlockSpec((1, B, G_CHUNK), lambda g: (0, 0, g)),
            pl.BlockSpec((1, B, G_CHUNK), lambda g: (1, 0, g)),
            pl.BlockSpec((1, B, G_CHUNK), lambda g: (2, 0, g)),
        ],
        out_specs=pl.BlockSpec((B * G_CHUNK,), lambda g: (g,)),
        out_shape=jax.ShapeDtypeStruct((B * NPTS,), jnp.int32),
    )(pts3, pts3, pts3)
    return idx


SCAT = 512


def _sc_voxel_body(idx_hbm, zeros_hbm, out_hbm,
                   hist, iba, ibb, ones, rbuf, obuf, lsem, ssem):
    c = lax.axis_index("c")
    s = lax.axis_index("s")
    wid = c * 16 + s

    pltpu.sync_copy(zeros_hbm.at[pl.ds(s * RED_WORDS, RED_WORDS)],
                    hist.at[pl.ds(s * RED_WORDS, RED_WORDS)])
    one = jnp.full((16,), 1.0, jnp.float32)
    for t in range(SCAT // 16):
        ones[pl.ds(t * 16, 16)] = one

    def chunk_off(k):
        q = 2 * s + k // 8
        return ((q // B_PER_SC) * (B * G_CHUNK)
                + (c * B_PER_SC + q % B_PER_SC) * G_CHUNK
                + (k % 8) * CHUNK_PTS)

    def load_desc(k, buf):
        return pltpu.make_async_copy(
            idx_hbm.at[pl.ds(chunk_off(k), CHUNK_PTS)], buf, lsem)

    def scatter(buf):
        copies = []
        for j in range(CHUNK_PTS // SCAT):
            copies.append(pltpu.async_copy(
                ones, hist.at[buf.at[pl.ds(j * SCAT, SCAT)]], ssem))
        for cp in copies:
            cp.wait()

    load_desc(0, iba).start()
    plsc.subcore_barrier()

    def chunk_body(m, _):
        load_desc(2 * m, iba).wait()
        load_desc(2 * m + 1, ibb).start()
        scatter(iba)
        load_desc(2 * m + 1, ibb).wait()

        @pl.when(m + 1 < N_CHUNKS // 2)
        def _():
            load_desc(2 * m + 2, iba).start()

        scatter(ibb)
        return 0

    lax.fori_loop(0, N_CHUNKS // 2, chunk_body, 0)
    plsc.subcore_barrier()

    acc = (jnp.zeros((16,), jnp.float32), jnp.zeros((16,), jnp.float32),
           jnp.zeros((16,), jnp.float32), jnp.zeros((16,), jnp.float32))
    for q in range(RED_WORDS // RED_CHUNK):
        pltpu.sync_copy(hist.at[pl.ds(s * RED_WORDS + q * RED_CHUNK, RED_CHUNK)],
                        rbuf)

        def red_body(r, a):
            a0, a1, a2, a3 = a
            base = r * 64
            a0 = a0 + rbuf[pl.ds(base, 16)]
            a1 = a1 + rbuf[pl.ds(base + 16, 16)]
            a2 = a2 + rbuf[pl.ds(base + 32, 16)]
            a3 = a3 + rbuf[pl.ds(base + 48, 16)]
            return (a0, a1, a2, a3)

        acc = lax.fori_loop(0, RED_CHUNK // 64, red_body, acc)

    for j in range(4):
        obuf[pl.ds(j * 16, 16)] = acc[j]
    pltpu.sync_copy(obuf, out_hbm.at[wid])


def _sc_partial_summaries(idx3, zeros):
    mesh = plsc.VectorSubcoreMesh(core_axis_name="c", subcore_axis_name="s")
    kern = functools.partial(
        pl.kernel,
        mesh=mesh,
        out_type=jax.ShapeDtypeStruct((NW, GRID), jnp.float32),
        scratch_types=[
            pltpu.VMEM_SHARED((HIST_N + 128,), jnp.float32),
            pltpu.VMEM((CHUNK_PTS,), jnp.int32),
            pltpu.VMEM((CHUNK_PTS,), jnp.int32),
            pltpu.VMEM((SCAT,), jnp.float32),
            pltpu.VMEM((RED_CHUNK,), jnp.float32),
            pltpu.VMEM((GRID,), jnp.float32),
            pltpu.SemaphoreType.DMA,
            pltpu.SemaphoreType.DMA,
        ],
    )(_sc_voxel_body)
    return kern(idx3, zeros)


def _interp(times, tq, feats):
    T = times.shape[0]
    idx = jnp.sum((times[None, :] < tq[:, None]).astype(jnp.int32), axis=1)
    idx = jnp.clip(idx, 1, T - 1)
    ii = lax.broadcasted_iota(jnp.int32, (TT, T), 1)
    oh0 = (ii == (idx - 1)[:, None]).astype(jnp.float32)
    oh1 = (ii == idx[:, None]).astype(jnp.float32)
    t0 = jnp.sum(oh0 * times[None, :], axis=1)
    t1 = jnp.sum(oh1 * times[None, :], axis=1)
    w = jnp.clip((tq - t0) / (t1 - t0 + 1e-8), 0.0, 1.0)
    M = oh0 * (1.0 - w)[:, None] + oh1 * w[:, None]
    return jnp.dot(M, feats, preferred_element_type=jnp.float32,
                   precision=lax.Precision.HIGHEST)


def _fuse_body(vision_ref, vt_ref, proprio_ref, pt_ref, imu_ref, it_ref,
               tt_ref, out_ref):
    tq = tt_ref[0, 0, :]
    v = _interp(vt_ref[0, 0, :], tq, vision_ref[0])
    p = _interp(pt_ref[0, 0, :], tq, proprio_ref[0])
    im = _interp(it_ref[0, 0, :], tq, imu_ref[0])
    out_ref[0] = jnp.concatenate([v, p, im], axis=-1)


def _assemble_body(fused_ref, part_ref, out_ref):
    summ = jnp.sum(part_ref[...], axis=0) * (1.0 / (B * GRID * GRID))
    sb = jnp.broadcast_to(summ[None, :], (TT, GRID))
    out_ref[0] = jnp.concatenate([fused_ref[0], sb], axis=-1)


def kernel(vision, vision_times, proprio, proprio_times, imu, imu_times,
           target_times, points):
    idx3 = _point_bin_indices(points)
    zeros = jnp.zeros((HIST_N,), jnp.float32)
    partials = _sc_partial_summaries(idx3, zeros)

    DF = DV + DP + DI
    fused = pl.pallas_call(
        _fuse_body,
        grid=(B,),
        in_specs=[
            pl.BlockSpec((1, TV, DV), lambda b: (b, 0, 0)),
            pl.BlockSpec((1, 1, TV), lambda b: (b, 0, 0)),
            pl.BlockSpec((1, TP, DP), lambda b: (b, 0, 0)),
            pl.BlockSpec((1, 1, TP), lambda b: (b, 0, 0)),
            pl.BlockSpec((1, TI, DI), lambda b: (b, 0, 0)),
            pl.BlockSpec((1, 1, TI), lambda b: (b, 0, 0)),
            pl.BlockSpec((1, 1, TT), lambda b: (b, 0, 0)),
        ],
        out_specs=pl.BlockSpec((1, TT, DF), lambda b: (b, 0, 0)),
        out_shape=jax.ShapeDtypeStruct((B, TT, DF), jnp.float32),
    )(vision, vision_times.reshape(B, 1, TV),
      proprio, proprio_times.reshape(B, 1, TP),
      imu, imu_times.reshape(B, 1, TI),
      target_times.reshape(B, 1, TT))

    out = pl.pallas_call(
        _assemble_body,
        grid=(B,),
        in_specs=[
            pl.BlockSpec((1, TT, DF), lambda b: (b, 0, 0)),
            pl.BlockSpec((NW, GRID), lambda b: (0, 0)),
        ],
        out_specs=pl.BlockSpec((1, TT, DF + GRID), lambda b: (b, 0, 0)),
        out_shape=jax.ShapeDtypeStruct((B, TT, DF + GRID), jnp.float32),
    )(fused, partials)
    return out


# speedup vs baseline: 1.0960x; 1.0960x over previous
import functools

import jax
import jax.numpy as jnp
from jax import lax
from jax.experimental import pallas as pl
from jax.experimental.pallas import tpu as pltpu
from jax.experimental.pallas import tpu_sc as plsc

B = 8
TV, DV = 64, 512
TP, DP = 256, 64
TI, DI = 512, 32
TT = 128
NPTS = 262144
GRID = 64
INV_VOX = 16.0
GMIN = -2.0

NW = 32
B_PER_SC = 4
HIST_N = B_PER_SC * GRID * GRID * GRID
DUMP = HIST_N
PTS_PER_TILE = (B_PER_SC * NPTS) // 16
CHUNK_PTS = 4096
N_CHUNKS = PTS_PER_TILE // CHUNK_PTS
RED_WORDS = HIST_N // 16
RED_CHUNK = 16384

G_CHUNK = 32768
N_G = NPTS // G_CHUNK


def _idx_body(x_ref, y_ref, z_ref, out_ref):
    def coords(ref):
        cf = (ref[0] - GMIN) * INV_VOX
        ci = jnp.minimum(jnp.maximum(cf, 0.0), 63.0).astype(jnp.int32)
        ok = (cf >= 0.0) & (cf < float(GRID))
        return ci, ok

    cx, okx = coords(x_ref)
    cy, oky = coords(y_ref)
    cz, okz = coords(z_ref)
    valid = okx & oky & okz
    bloc = lax.broadcasted_iota(jnp.int32, (B, G_CHUNK), 0) % B_PER_SC
    spread = lax.broadcasted_iota(jnp.int32, (B, G_CHUNK), 1) % 128
    flat = bloc * (GRID * GRID * GRID) + cx * (GRID * GRID) + cy * GRID + cz
    idx = jnp.where(valid, flat, DUMP + spread)
    out_ref[...] = idx.reshape(B * G_CHUNK)


def _point_bin_indices(points):
    pts3 = jnp.transpose(points, (2, 0, 1))
    idx = pl.pallas_call(
        _idx_body,
        grid=(N_G,),
        in_specs=[
            pl.BlockSpec((1, B, G_CHUNK), lambda g: (0, 0, g)),
            pl.BlockSpec((1, B, G_CHUNK), lambda g: (1, 0, g)),
            pl.BlockSpec((1, B, G_CHUNK), lambda g: (2, 0, g)),
        ],
        out_specs=pl.BlockSpec((B * G_CHUNK,), lambda g: (g,)),
        out_shape=jax.ShapeDtypeStruct((B * NPTS,), jnp.int32),
    )(pts3, pts3, pts3)
    return idx


SCAT = 512


def _sc_voxel_body(idx_hbm, out_hbm,
                   hist, iba, ibb, ones, rbuf, obuf, lsem, ssem):
    c = lax.axis_index("c")
    s = lax.axis_index("s")
    wid = c * 16 + s

    zero16 = jnp.zeros((16,), jnp.float32)

    @plsc.parallel_loop(0, RED_CHUNK // 16, unroll=8)
    def _zero(t):
        rbuf[pl.ds(t * 16, 16)] = zero16

    one = jnp.full((16,), 1.0, jnp.float32)
    for t in range(SCAT // 16):
        ones[pl.ds(t * 16, 16)] = one
    for q in range(RED_WORDS // RED_CHUNK):
        pltpu.sync_copy(rbuf,
                        hist.at[pl.ds(s * RED_WORDS + q * RED_CHUNK,
                                      RED_CHUNK)])

    def chunk_off(k):
        q = 2 * s + k // 8
        return ((q // B_PER_SC) * (B * G_CHUNK)
                + (c * B_PER_SC + q % B_PER_SC) * G_CHUNK
                + (k % 8) * CHUNK_PTS)

    def load_desc(k, buf):
        return pltpu.make_async_copy(
            idx_hbm.at[pl.ds(chunk_off(k), CHUNK_PTS)], buf, lsem)

    def scatter(buf):
        copies = []
        for j in range(CHUNK_PTS // SCAT):
            copies.append(pltpu.async_copy(
                ones, hist.at[buf.at[pl.ds(j * SCAT, SCAT)]], ssem))
        for cp in copies:
            cp.wait()

    load_desc(0, iba).start()
    plsc.subcore_barrier()

    def chunk_body(m, _):
        load_desc(2 * m, iba).wait()
        load_desc(2 * m + 1, ibb).start()
        scatter(iba)
        load_desc(2 * m + 1, ibb).wait()

        @pl.when(m + 1 < N_CHUNKS // 2)
        def _():
            load_desc(2 * m + 2, iba).start()

        scatter(ibb)
        return 0

    lax.fori_loop(0, N_CHUNKS // 2, chunk_body, 0)
    plsc.subcore_barrier()

    acc = (jnp.zeros((16,), jnp.float32), jnp.zeros((16,), jnp.float32),
           jnp.zeros((16,), jnp.float32), jnp.zeros((16,), jnp.float32))
    for q in range(RED_WORDS // RED_CHUNK):
        pltpu.sync_copy(hist.at[pl.ds(s * RED_WORDS + q * RED_CHUNK, RED_CHUNK)],
                        rbuf)

        @plsc.parallel_loop(0, RED_CHUNK // 64, unroll=4, carry=acc)
        def red_body(r, a):
            a0, a1, a2, a3 = a
            base = r * 64
            a0 = a0 + rbuf[pl.ds(base, 16)]
            a1 = a1 + rbuf[pl.ds(base + 16, 16)]
            a2 = a2 + rbuf[pl.ds(base + 32, 16)]
            a3 = a3 + rbuf[pl.ds(base + 48, 16)]
            return (a0, a1, a2, a3)

        acc = red_body

    for j in range(4):
        obuf[pl.ds(j * 16, 16)] = acc[j]
    pltpu.sync_copy(obuf, out_hbm.at[wid])


def _sc_partial_summaries(idx3):
    mesh = plsc.VectorSubcoreMesh(core_axis_name="c", subcore_axis_name="s")
    kern = functools.partial(
        pl.kernel,
        mesh=mesh,
        out_type=jax.ShapeDtypeStruct((NW, GRID), jnp.float32),
        scratch_types=[
            pltpu.VMEM_SHARED((HIST_N + 128,), jnp.float32),
            pltpu.VMEM((CHUNK_PTS,), jnp.int32),
            pltpu.VMEM((CHUNK_PTS,), jnp.int32),
            pltpu.VMEM((SCAT,), jnp.float32),
            pltpu.VMEM((RED_CHUNK,), jnp.float32),
            pltpu.VMEM((GRID,), jnp.float32),
            pltpu.SemaphoreType.DMA,
            pltpu.SemaphoreType.DMA,
        ],
    )(_sc_voxel_body)
    return kern(idx3)


def _interp(times, tq, feats):
    T = times.shape[0]
    idx = jnp.sum((times[None, :] < tq[:, None]).astype(jnp.int32), axis=1)
    idx = jnp.clip(idx, 1, T - 1)
    ii = lax.broadcasted_iota(jnp.int32, (TT, T), 1)
    oh0 = (ii == (idx - 1)[:, None]).astype(jnp.float32)
    oh1 = (ii == idx[:, None]).astype(jnp.float32)
    t0 = jnp.sum(oh0 * times[None, :], axis=1)
    t1 = jnp.sum(oh1 * times[None, :], axis=1)
    w = jnp.clip((tq - t0) / (t1 - t0 + 1e-8), 0.0, 1.0)
    M = oh0 * (1.0 - w)[:, None] + oh1 * w[:, None]
    return jnp.dot(M, feats, preferred_element_type=jnp.float32,
                   precision=lax.Precision.HIGHEST)


def _fuse_body(vision_ref, vt_ref, proprio_ref, pt_ref, imu_ref, it_ref,
               tt_ref, out_ref):
    tq = tt_ref[0, 0, :]
    v = _interp(vt_ref[0, 0, :], tq, vision_ref[0])
    p = _interp(pt_ref[0, 0, :], tq, proprio_ref[0])
    im = _interp(it_ref[0, 0, :], tq, imu_ref[0])
    out_ref[0] = jnp.concatenate([v, p, im], axis=-1)


def _summ_body(part_ref, out_ref):
    summ = jnp.sum(part_ref[...], axis=0) * (1.0 / (B * GRID * GRID))
    out_ref[0] = jnp.broadcast_to(summ[None, :], (TT, GRID))


def kernel(vision, vision_times, proprio, proprio_times, imu, imu_times,
           target_times, points):
    idx3 = _point_bin_indices(points)
    partials = _sc_partial_summaries(idx3)

    DF = DV + DP + DI
    fused = pl.pallas_call(
        _fuse_body,
        grid=(B,),
        in_specs=[
            pl.BlockSpec((1, TV, DV), lambda b: (b, 0, 0)),
            pl.BlockSpec((1, 1, TV), lambda b: (b, 0, 0)),
            pl.BlockSpec((1, TP, DP), lambda b: (b, 0, 0)),
            pl.BlockSpec((1, 1, TP), lambda b: (b, 0, 0)),
            pl.BlockSpec((1, TI, DI), lambda b: (b, 0, 0)),
            pl.BlockSpec((1, 1, TI), lambda b: (b, 0, 0)),
            pl.BlockSpec((1, 1, TT), lambda b: (b, 0, 0)),
        ],
        out_specs=pl.BlockSpec((1, TT, DF), lambda b: (b, 0, 0)),
        out_shape=jax.ShapeDtypeStruct((B, TT, DF), jnp.float32),
    )(vision, vision_times.reshape(B, 1, TV),
      proprio, proprio_times.reshape(B, 1, TP),
      imu, imu_times.reshape(B, 1, TI),
      target_times.reshape(B, 1, TT))

    vox = pl.pallas_call(
        _summ_body,
        grid=(B,),
        in_specs=[pl.BlockSpec((NW, GRID), lambda b: (0, 0))],
        out_specs=pl.BlockSpec((1, TT, GRID), lambda b: (b, 0, 0)),
        out_shape=jax.ShapeDtypeStruct((B, TT, GRID), jnp.float32),
    )(partials)
    return jnp.concatenate([fused, vox], axis=-1)
